# baseline (speedup 1.0000x reference)
import jax
import jax.numpy as jnp
from jax.experimental import pallas as pl
from jax.experimental.pallas import tpu as pltpu

_TOPK = 64
_RB = 512
_CB = 1024
_NC = 16384 // _CB
_SB = 128
_MAXI = 24


def _search_slice(acc_ref, thr_ref, mx_ref, lo_ref, hi_ref, dn_ref, r):
    ee = acc_ref[pl.ds(r * _SB, _SB), :]
    aa = jnp.abs(ee)
    zeros = jnp.zeros((_SB, 128), jnp.float32)
    lo_ref[...] = zeros
    hi_ref[...] = mx_ref[pl.ds(r * _SB, _SB), :]
    dn_ref[...] = zeros
    thr_ref[pl.ds(r * _SB, _SB), :] = zeros

    a16 = aa.astype(jnp.bfloat16)

    def citer(i, c):
        lo = lo_ref[:, :1]
        hi = hi_ref[:, :1]
        mid = 0.5 * (lo + hi)
        m16 = mid.astype(jnp.bfloat16)
        m32 = m16.astype(jnp.float32)
        sel = jnp.where(a16 >= m16, jnp.bfloat16(1.0), jnp.bfloat16(0.0))
        part = jnp.sum(sel.reshape(_SB, 128, 128), axis=1)
        cnt = jnp.sum(part.astype(jnp.float32), axis=1, keepdims=True)
        ge = cnt >= _TOPK
        lo_ref[...] = jnp.broadcast_to(
            jnp.where(ge, m32 * (1.0 - 2.0 ** -8), lo), (_SB, 128))
        hi_ref[...] = jnp.broadcast_to(jnp.where(ge, hi, m32), (_SB, 128))
        return c

    jax.lax.fori_loop(0, 9, citer, 0)

    def biter(i, c):
        @pl.when(jnp.any(dn_ref[...] == 0.0))
        def _():
            lo = lo_ref[:, :1]
            hi = hi_ref[:, :1]
            dn = dn_ref[:, :1]
            t = thr_ref[pl.ds(r * _SB, _SB), :1]
            mid = 0.5 * (lo + hi)
            cnt = jnp.sum(jnp.where(aa >= mid, 1.0, 0.0), axis=1,
                          keepdims=True)
            live = dn == 0.0
            hit = jnp.logical_and(cnt == _TOPK, live)
            ge = cnt >= _TOPK
            thr_ref[pl.ds(r * _SB, _SB), :] = jnp.broadcast_to(
                jnp.where(hit, mid, t), (_SB, 128))
            dn_ref[...] = jnp.broadcast_to(jnp.where(hit, 1.0, dn),
                                           (_SB, 128))
            adv = jnp.logical_and(live, jnp.logical_not(hit))
            lo_ref[...] = jnp.broadcast_to(
                jnp.where(jnp.logical_and(adv, ge), mid, lo), (_SB, 128))
            hi_ref[...] = jnp.broadcast_to(
                jnp.where(jnp.logical_and(adv, jnp.logical_not(ge)),
                          mid, hi), (_SB, 128))
        return c

    jax.lax.fori_loop(0, _MAXI, biter, 0)

    @pl.when(jnp.any(dn_ref[...] == 0.0))
    def _():
        abits = jax.lax.bitcast_convert_type(aa, jnp.int32)

        def bbody(i, tb):
            cand = tb | jnp.left_shift(jnp.int32(1), 30 - i)
            cntb = jnp.sum((abits >= cand).astype(jnp.int32), axis=1,
                           keepdims=True)
            return jnp.where(cntb >= _TOPK, cand, tb)

        tb = jax.lax.fori_loop(0, 31, bbody,
                               jnp.zeros((_SB, 1), jnp.int32))
        tb_f = jax.lax.bitcast_convert_type(tb, jnp.float32)
        thr_ref[pl.ds(r * _SB, _SB), :] = jnp.where(
            dn_ref[...] == 0.0,
            jnp.broadcast_to(tb_f, (_SB, 128)),
            thr_ref[pl.ds(r * _SB, _SB), :])


def _enc_kernel(x_ref, w_ref, b_ref, o_ref,
                acc_ref, thr_ref, mx_ref, lo_ref, hi_ref, dn_ref):
    j = pl.program_id(1)

    @pl.when(j < _NC)
    def _():
        x = x_ref[...]
        xn = x / jnp.sqrt(jnp.sum(x * x, axis=1, keepdims=True))
        e = jnp.dot(xn, w_ref[...], preferred_element_type=jnp.float32)
        e = e + b_ref[...]
        acc_ref[:, pl.ds(j * _CB, _CB)] = e
        cmx = jnp.max(jnp.abs(e), axis=1, keepdims=True)
        prev = jnp.where(j == 0, 0.0, mx_ref[:, :1])
        mx_ref[...] = jnp.broadcast_to(jnp.maximum(prev, cmx),
                                       (_RB, 128))

    @pl.when(j == _NC - 1)
    def _():
        def row_slice(r, c):
            _search_slice(acc_ref, thr_ref, mx_ref,
                          lo_ref, hi_ref, dn_ref, r)
            return c

        jax.lax.fori_loop(0, _RB // _SB, row_slice, 0)

    @pl.when(j >= _NC)
    def _():
        c = j - _NC
        chunk = acc_ref[:, pl.ds(c * _CB, _CB)]
        tt = thr_ref[:, :1]
        o_ref[...] = jnp.where(jnp.abs(chunk) >= tt, chunk, 0.0)


def kernel(x, W, b):
    M, Kd = x.shape
    N = W.shape[1]
    b2 = b.reshape(1, N)
    grid = (M // _RB, 2 * _NC)
    return pl.pallas_call(
        _enc_kernel,
        grid=grid,
        in_specs=[
            pl.BlockSpec((_RB, Kd), lambda i, j: (i, 0)),
            pl.BlockSpec((Kd, _CB), lambda i, j: (0, jnp.minimum(j, _NC - 1))),
            pl.BlockSpec((1, _CB), lambda i, j: (0, jnp.minimum(j, _NC - 1))),
        ],
        out_specs=pl.BlockSpec(
            (_RB, _CB),
            lambda i, j: (i, jnp.clip(j - _NC, 0, _NC - 1))),
        out_shape=jax.ShapeDtypeStruct((M, N), jnp.float32),
        scratch_shapes=[
            pltpu.VMEM((_RB, 16384), jnp.float32),
            pltpu.VMEM((_RB, 128), jnp.float32),
            pltpu.VMEM((_RB, 128), jnp.float32),
            pltpu.VMEM((_SB, 128), jnp.float32),
            pltpu.VMEM((_SB, 128), jnp.float32),
            pltpu.VMEM((_SB, 128), jnp.float32),
        ],
        compiler_params=pltpu.CompilerParams(
            dimension_semantics=("parallel", "arbitrary"),
        ),
    )(x, W, b2)


# speedup vs baseline: 1.2698x; 1.2698x over previous
import jax
import jax.numpy as jnp
from jax.experimental import pallas as pl
from jax.experimental.pallas import tpu as pltpu

_TOPK = 64
_RB = 512
_CB = 1024
_NC = 16384 // _CB
_SB = 128
_MAXI = 24


def _search_slice(acc_ref, thr_ref, mx_ref, lo_ref, hi_ref, dn_ref, r):
    ee = acc_ref[pl.ds(r * _SB, _SB), :]
    aa = jnp.abs(ee)
    zeros = jnp.zeros((_SB, 128), jnp.float32)
    lo_ref[...] = zeros
    hi_ref[...] = mx_ref[pl.ds(r * _SB, _SB), :]
    dn_ref[...] = zeros
    thr_ref[pl.ds(r * _SB, _SB), :] = zeros

    def biter(i, c):
        @pl.when(jnp.any(dn_ref[...] == 0.0))
        def _():
            lo = lo_ref[:, :1]
            hi = hi_ref[:, :1]
            dn = dn_ref[:, :1]
            t = thr_ref[pl.ds(r * _SB, _SB), :1]
            mid = 0.5 * (lo + hi)
            cnt = jnp.sum(jnp.where(aa >= mid, 1.0, 0.0), axis=1,
                          keepdims=True)
            live = dn == 0.0
            hit = jnp.logical_and(cnt == _TOPK, live)
            ge = cnt >= _TOPK
            thr_ref[pl.ds(r * _SB, _SB), :] = jnp.broadcast_to(
                jnp.where(hit, mid, t), (_SB, 128))
            dn_ref[...] = jnp.broadcast_to(jnp.where(hit, 1.0, dn),
                                           (_SB, 128))
            adv = jnp.logical_and(live, jnp.logical_not(hit))
            lo_ref[...] = jnp.broadcast_to(
                jnp.where(jnp.logical_and(adv, ge), mid, lo), (_SB, 128))
            hi_ref[...] = jnp.broadcast_to(
                jnp.where(jnp.logical_and(adv, jnp.logical_not(ge)),
                          mid, hi), (_SB, 128))
        return c

    jax.lax.fori_loop(0, _MAXI, biter, 0)

    @pl.when(jnp.any(dn_ref[...] == 0.0))
    def _():
        abits = jax.lax.bitcast_convert_type(aa, jnp.int32)

        def bbody(i, tb):
            cand = tb | jnp.left_shift(jnp.int32(1), 30 - i)
            cntb = jnp.sum((abits >= cand).astype(jnp.int32), axis=1,
                           keepdims=True)
            return jnp.where(cntb >= _TOPK, cand, tb)

        tb = jax.lax.fori_loop(0, 31, bbody,
                               jnp.zeros((_SB, 1), jnp.int32))
        tb_f = jax.lax.bitcast_convert_type(tb, jnp.float32)
        thr_ref[pl.ds(r * _SB, _SB), :] = jnp.where(
            dn_ref[...] == 0.0,
            jnp.broadcast_to(tb_f, (_SB, 128)),
            thr_ref[pl.ds(r * _SB, _SB), :])


def _enc_kernel(x_ref, w_ref, b_ref, o_ref,
                acc_ref, thr_ref, mx_ref, lo_ref, hi_ref, dn_ref):
    j = pl.program_id(1)

    @pl.when(j < _NC)
    def _():
        x = x_ref[...]
        xn = x / jnp.sqrt(jnp.sum(x * x, axis=1, keepdims=True))
        e = jnp.dot(xn, w_ref[...], preferred_element_type=jnp.float32)
        e = e + b_ref[...]
        acc_ref[:, pl.ds(j * _CB, _CB)] = e
        cmx = jnp.max(jnp.abs(e), axis=1, keepdims=True)
        prev = jnp.where(j == 0, 0.0, mx_ref[:, :1])
        mx_ref[...] = jnp.broadcast_to(jnp.maximum(prev, cmx),
                                       (_RB, 128))

    @pl.when(j == _NC - 1)
    def _():
        def row_slice(r, c):
            _search_slice(acc_ref, thr_ref, mx_ref,
                          lo_ref, hi_ref, dn_ref, r)
            return c

        jax.lax.fori_loop(0, _RB // _SB, row_slice, 0)

    @pl.when(j >= _NC)
    def _():
        c = j - _NC
        chunk = acc_ref[:, pl.ds(c * _CB, _CB)]
        tt = thr_ref[:, :1]
        o_ref[...] = jnp.where(jnp.abs(chunk) >= tt, chunk, 0.0)


def kernel(x, W, b):
    M, Kd = x.shape
    N = W.shape[1]
    b2 = b.reshape(1, N)
    grid = (M // _RB, 2 * _NC)
    return pl.pallas_call(
        _enc_kernel,
        grid=grid,
        in_specs=[
            pl.BlockSpec((_RB, Kd), lambda i, j: (i, 0)),
            pl.BlockSpec((Kd, _CB), lambda i, j: (0, jnp.minimum(j, _NC - 1))),
            pl.BlockSpec((1, _CB), lambda i, j: (0, jnp.minimum(j, _NC - 1))),
        ],
        out_specs=pl.BlockSpec(
            (_RB, _CB),
            lambda i, j: (i, jnp.clip(j - _NC, 0, _NC - 1))),
        out_shape=jax.ShapeDtypeStruct((M, N), jnp.float32),
        scratch_shapes=[
            pltpu.VMEM((_RB, 16384), jnp.float32),
            pltpu.VMEM((_RB, 128), jnp.float32),
            pltpu.VMEM((_RB, 128), jnp.float32),
            pltpu.VMEM((_SB, 128), jnp.float32),
            pltpu.VMEM((_SB, 128), jnp.float32),
            pltpu.VMEM((_SB, 128), jnp.float32),
        ],
        compiler_params=pltpu.CompilerParams(
            dimension_semantics=("parallel", "arbitrary"),
        ),
    )(x, W, b2)


# speedup vs baseline: 1.2831x; 1.0105x over previous
import jax
import jax.numpy as jnp
from jax.experimental import pallas as pl
from jax.experimental.pallas import tpu as pltpu

_TOPK = 64
_RB = 512
_CB = 1024
_NC = 16384 // _CB
_SB = 128
_MAXI = 24


def _search_slice(acc_ref, thr_ref, mx_ref, lo_ref, hi_ref, dn_ref, r):
    ee = acc_ref[pl.ds(r * _SB, _SB), :]
    aa = jnp.abs(ee)
    zeros = jnp.zeros((_SB, 128), jnp.float32)
    lo_ref[...] = zeros
    hi_ref[...] = mx_ref[pl.ds(r * _SB, _SB), :]
    dn_ref[...] = zeros
    thr_ref[pl.ds(r * _SB, _SB), :] = zeros

    def biter(i, c):
        @pl.when(jnp.any(dn_ref[...] == 0.0))
        def _():
            lo = lo_ref[:, :1]
            hi = hi_ref[:, :1]
            dn = dn_ref[:, :1]
            mid = 0.5 * (lo + hi)
            cnt = jnp.sum(jnp.where(aa >= mid, 1.0, 0.0), axis=1,
                          keepdims=True)
            live = dn == 0.0
            hit = jnp.logical_and(cnt == _TOPK, live)
            ge = cnt >= _TOPK
            dn_ref[...] = jnp.broadcast_to(jnp.where(hit, 1.0, dn),
                                           (_SB, 128))
            lo_ref[...] = jnp.broadcast_to(
                jnp.where(jnp.logical_and(live, ge), mid, lo), (_SB, 128))
            hi_ref[...] = jnp.broadcast_to(
                jnp.where(jnp.logical_and(
                    live, jnp.logical_or(hit, jnp.logical_not(ge))),
                    mid, hi), (_SB, 128))
        return c

    jax.lax.fori_loop(0, _MAXI, biter, 0)
    thr_ref[pl.ds(r * _SB, _SB), :] = lo_ref[...]

    @pl.when(jnp.any(dn_ref[...] == 0.0))
    def _():
        abits = jax.lax.bitcast_convert_type(aa, jnp.int32)

        def bbody(i, tb):
            cand = tb | jnp.left_shift(jnp.int32(1), 30 - i)
            cntb = jnp.sum((abits >= cand).astype(jnp.int32), axis=1,
                           keepdims=True)
            return jnp.where(cntb >= _TOPK, cand, tb)

        tb = jax.lax.fori_loop(0, 31, bbody,
                               jnp.zeros((_SB, 1), jnp.int32))
        tb_f = jax.lax.bitcast_convert_type(tb, jnp.float32)
        thr_ref[pl.ds(r * _SB, _SB), :] = jnp.where(
            dn_ref[...] == 0.0,
            jnp.broadcast_to(tb_f, (_SB, 128)),
            thr_ref[pl.ds(r * _SB, _SB), :])


def _enc_kernel(x_ref, w_ref, b_ref, o_ref,
                acc_ref, thr_ref, mx_ref, lo_ref, hi_ref, dn_ref):
    j = pl.program_id(1)

    @pl.when(j < _NC)
    def _():
        x = x_ref[...]
        xn = x / jnp.sqrt(jnp.sum(x * x, axis=1, keepdims=True))
        e = jnp.dot(xn, w_ref[...], preferred_element_type=jnp.float32)
        e = e + b_ref[...]
        acc_ref[:, pl.ds(j * _CB, _CB)] = e
        cmx = jnp.max(jnp.abs(e), axis=1, keepdims=True)
        prev = jnp.where(j == 0, 0.0, mx_ref[:, :1])
        mx_ref[...] = jnp.broadcast_to(jnp.maximum(prev, cmx),
                                       (_RB, 128))

    @pl.when(j == _NC - 1)
    def _():
        def row_slice(r, c):
            _search_slice(acc_ref, thr_ref, mx_ref,
                          lo_ref, hi_ref, dn_ref, r)
            return c

        jax.lax.fori_loop(0, _RB // _SB, row_slice, 0)

    @pl.when(j >= _NC)
    def _():
        c = j - _NC
        chunk = acc_ref[:, pl.ds(c * _CB, _CB)]
        tt = thr_ref[:, :1]
        o_ref[...] = jnp.where(jnp.abs(chunk) >= tt, chunk, 0.0)


def kernel(x, W, b):
    M, Kd = x.shape
    N = W.shape[1]
    b2 = b.reshape(1, N)
    grid = (M // _RB, 2 * _NC)
    return pl.pallas_call(
        _enc_kernel,
        grid=grid,
        in_specs=[
            pl.BlockSpec((_RB, Kd), lambda i, j: (i, 0)),
            pl.BlockSpec((Kd, _CB), lambda i, j: (0, jnp.minimum(j, _NC - 1))),
            pl.BlockSpec((1, _CB), lambda i, j: (0, jnp.minimum(j, _NC - 1))),
        ],
        out_specs=pl.BlockSpec(
            (_RB, _CB),
            lambda i, j: (i, jnp.clip(j - _NC, 0, _NC - 1))),
        out_shape=jax.ShapeDtypeStruct((M, N), jnp.float32),
        scratch_shapes=[
            pltpu.VMEM((_RB, 16384), jnp.float32),
            pltpu.VMEM((_RB, 128), jnp.float32),
            pltpu.VMEM((_RB, 128), jnp.float32),
            pltpu.VMEM((_SB, 128), jnp.float32),
            pltpu.VMEM((_SB, 128), jnp.float32),
            pltpu.VMEM((_SB, 128), jnp.float32),
        ],
        compiler_params=pltpu.CompilerParams(
            dimension_semantics=("parallel", "arbitrary"),
        ),
    )(x, W, b2)
